# Initial kernel scaffold; baseline (speedup 1.0000x reference)
import jax
import jax.numpy as jnp
from jax.experimental import pallas as pl


def kernel(x, edge_index, W_self, b_self, W_neigh, b_neigh):
    raise NotImplementedError("write your pallas kernel here")


# speedup vs baseline: 3.7779x; 3.7779x over previous
import functools

import jax
import jax.numpy as jnp
from jax import lax
from jax.experimental import pallas as pl
from jax.experimental.pallas import tpu as pltpu
from jax.experimental.pallas import tpu_sc as plsc

N = 10000
E = 320000
D = 128
DH = D // 2

NC = 2
NS = 16
CH = 128
K = 160
G = 16
NG = K // G
E_PAD = NS * K * CH
N_PAD = 10240
STRIPE = N_PAD // NS
DW = 8


def _sc_aggregate(x2, src2, dst_r, za, zd, ones):
    mesh = plsc.VectorSubcoreMesh(core_axis_name="c", subcore_axis_name="s")

    @functools.partial(
        pl.kernel,
        out_type=[
            jax.ShapeDtypeStruct((NC, N_PAD, DH), jnp.float32),
            jax.ShapeDtypeStruct((N_PAD, DW), jnp.float32),
        ],
        mesh=mesh,
        scratch_types=[
            pltpu.VMEM((G, CH), jnp.int32),
            pltpu.VMEM((G, CH), jnp.int32),
            pltpu.VMEM((CH, DH), jnp.float32),
            pltpu.VMEM((CH, DW), jnp.float32),
            pltpu.VMEM((CH, DW), jnp.float32),
            pltpu.VMEM_SHARED((N_PAD, DH), jnp.float32),
            pltpu.VMEM_SHARED((N_PAD, DW), jnp.float32),
            pltpu.SemaphoreType.DMA,
        ],
        compiler_params=pltpu.CompilerParams(use_tc_tiling_on_sc=False),
    )
    def run(x_hbm, src_hbm, dst_hbm, za_hbm, zd_hbm, ones_hbm,
            agg_out, deg_out,
            src_v, dst_v, rows_v, ones_v, degst_v, agg_sh, deg_sh, sem):
        c = lax.axis_index("c")
        s = lax.axis_index("s")

        pltpu.sync_copy(za_hbm, rows_v)
        pltpu.sync_copy(zd_hbm, degst_v)
        for t in range(STRIPE // CH):
            off = s * STRIPE + t * CH
            pltpu.sync_copy(rows_v, agg_sh.at[pl.ds(off, CH)])
            pltpu.sync_copy(degst_v, deg_sh.at[pl.ds(off, CH)])
        pltpu.sync_copy(ones_hbm, ones_v)

        plsc.subcore_barrier()

        def group(g, carry):
            pltpu.sync_copy(src_hbm.at[c, s, pl.ds(g * G, G)], src_v)
            pltpu.sync_copy(dst_hbm.at[s, pl.ds(g * G, G)], dst_v)

            def chunk(j, carry2):
                pltpu.async_copy(x_hbm.at[src_v.at[j]], rows_v, sem).wait()
                pltpu.sync_copy(rows_v, agg_sh.at[dst_v.at[j]], add=True)

                @pl.when(c == 0)
                def _():
                    pltpu.sync_copy(ones_v, deg_sh.at[dst_v.at[j]], add=True)

                return carry2

            return lax.fori_loop(0, G, chunk, carry)

        lax.fori_loop(0, NG, group, 0)

        plsc.subcore_barrier()

        for t in range(STRIPE // CH):
            off = s * STRIPE + t * CH
            pltpu.sync_copy(agg_sh.at[pl.ds(off, CH)], rows_v)
            pltpu.sync_copy(rows_v, agg_out.at[c, pl.ds(off, CH)])

        @pl.when(c == 0)
        def _():
            for t in range(STRIPE // CH):
                off = s * STRIPE + t * CH
                pltpu.sync_copy(deg_sh.at[pl.ds(off, CH)], degst_v)
                pltpu.sync_copy(degst_v, deg_out.at[pl.ds(off, CH)])

    return run(x2, src2, dst_r, za, zd, ones)


def _tc_body(x_ref, a_ref, d_ref, ws_ref, wn_ref, bs_ref, bn_ref, o_ref):
    xb = x_ref[...]
    deg = jnp.maximum(d_ref[...], 1.0)
    aggn = a_ref[...] / deg
    h = (jnp.dot(xb, ws_ref[...], preferred_element_type=jnp.float32)
         + jnp.dot(aggn, wn_ref[...], preferred_element_type=jnp.float32)
         + bs_ref[...] + bn_ref[...])
    o_ref[...] = jnp.maximum(h, 0.0)


def kernel(x, edge_index, W_self, b_self, W_neigh, b_neigh):
    src = edge_index[0].astype(jnp.int32)
    dst = edge_index[1].astype(jnp.int32)
    pad_e = E_PAD - E
    src = jnp.concatenate([src, jnp.zeros((pad_e,), jnp.int32)])
    pad_dst = N + (jnp.arange(pad_e, dtype=jnp.int32) % (N_PAD - N))
    dst = jnp.concatenate([dst, pad_dst])
    src_r = src.reshape(NS, K, CH)
    dst_r = dst.reshape(NS, K, CH)
    src2 = jnp.stack([src_r, src_r + N_PAD])
    x_pad = jnp.concatenate([x, jnp.zeros((N_PAD - N, D), jnp.float32)])
    x2 = jnp.concatenate([x_pad[:, :DH], x_pad[:, DH:]], axis=0)
    za = jnp.zeros((CH, DH), jnp.float32)
    zd = jnp.zeros((CH, DW), jnp.float32)
    ones = jnp.ones((CH, DW), jnp.float32)

    agg_p, deg_p = _sc_aggregate(x2, src2, dst_r, za, zd, ones)
    agg = jnp.concatenate([agg_p[0], agg_p[1]], axis=1)
    deg1 = deg_p[:, :1]

    R = 1024
    h = pl.pallas_call(
        _tc_body,
        grid=(N_PAD // R,),
        in_specs=[
            pl.BlockSpec((R, D), lambda i: (i, 0)),
            pl.BlockSpec((R, D), lambda i: (i, 0)),
            pl.BlockSpec((R, 1), lambda i: (i, 0)),
            pl.BlockSpec((D, D), lambda i: (0, 0)),
            pl.BlockSpec((D, D), lambda i: (0, 0)),
            pl.BlockSpec((1, D), lambda i: (0, 0)),
            pl.BlockSpec((1, D), lambda i: (0, 0)),
        ],
        out_specs=pl.BlockSpec((R, D), lambda i: (i, 0)),
        out_shape=jax.ShapeDtypeStruct((N_PAD, D), jnp.float32),
    )(x_pad, agg, deg1, W_self.T, W_neigh.T,
      b_self.reshape(1, D), b_neigh.reshape(1, D))
    return h[:N]


# speedup vs baseline: 4.9474x; 1.3096x over previous
import functools

import jax
import jax.numpy as jnp
from jax import lax
from jax.experimental import pallas as pl
from jax.experimental.pallas import tpu as pltpu
from jax.experimental.pallas import tpu_sc as plsc

N = 10000
E = 320000
D = 128
DH = D // 2

NC = 2
NS = 16
CH = 128
K = 160
NBUF = 4
E_PAD = NS * K * CH
N_PAD = 10240
STRIPE = N_PAD // NS
DW = 8


def _sc_aggregate(x2, src2, dst_r, za, zd, ones):
    mesh = plsc.VectorSubcoreMesh(core_axis_name="c", subcore_axis_name="s")

    @functools.partial(
        pl.kernel,
        out_type=[
            jax.ShapeDtypeStruct((NC, N_PAD, DH), jnp.float32),
            jax.ShapeDtypeStruct((N_PAD, DW), jnp.float32),
        ],
        mesh=mesh,
        scratch_types=[
            pltpu.VMEM((K, CH), jnp.int32),
            pltpu.VMEM((K, CH), jnp.int32),
            pltpu.VMEM((NBUF, CH, DH), jnp.float32),
            pltpu.VMEM((CH, DW), jnp.float32),
            pltpu.VMEM((CH, DW), jnp.float32),
            pltpu.VMEM_SHARED((N_PAD, DH), jnp.float32),
            pltpu.VMEM_SHARED((N_PAD, DW), jnp.float32),
            pltpu.SemaphoreType.DMA((NBUF,)),
            pltpu.SemaphoreType.DMA((NBUF,)),
            pltpu.SemaphoreType.DMA,
        ],
        compiler_params=pltpu.CompilerParams(use_tc_tiling_on_sc=False),
    )
    def run(x_hbm, src_hbm, dst_hbm, za_hbm, zd_hbm, ones_hbm,
            agg_out, deg_out,
            src_v, dst_v, rows_v, ones_v, degst_v, agg_sh, deg_sh,
            gsem, ssem, dsem):
        c = lax.axis_index("c")
        s = lax.axis_index("s")

        pltpu.sync_copy(za_hbm, rows_v.at[0])
        pltpu.sync_copy(zd_hbm, degst_v)
        for t in range(STRIPE // CH):
            off = s * STRIPE + t * CH
            pltpu.sync_copy(rows_v.at[0], agg_sh.at[pl.ds(off, CH)])
            pltpu.sync_copy(degst_v, deg_sh.at[pl.ds(off, CH)])
        pltpu.sync_copy(ones_hbm, ones_v)

        pltpu.sync_copy(src_hbm.at[c, s], src_v)
        pltpu.sync_copy(dst_hbm.at[s], dst_v)

        plsc.subcore_barrier()

        for b in range(NBUF):
            pltpu.async_copy(x_hbm.at[src_v.at[b]], rows_v.at[b], gsem.at[b])

        def step(i, carry):
            for b in range(NBUF):
                ck = i * NBUF + b
                pltpu.make_async_copy(
                    x_hbm.at[src_v.at[ck]], rows_v.at[b], gsem.at[b]).wait()
                pltpu.async_copy(
                    rows_v.at[b], agg_sh.at[dst_v.at[ck]], ssem.at[b],
                    add=True)

                @pl.when(c == 0)
                def _():
                    pltpu.async_copy(
                        ones_v, deg_sh.at[dst_v.at[ck]], dsem, add=True)

                @pl.when(ck + NBUF < K)
                def _():
                    pltpu.make_async_copy(
                        rows_v.at[b], agg_sh.at[dst_v.at[ck]],
                        ssem.at[b]).wait()
                    pltpu.async_copy(
                        x_hbm.at[src_v.at[ck + NBUF]], rows_v.at[b],
                        gsem.at[b])

            return carry

        lax.fori_loop(0, K // NBUF, step, 0)

        for b in range(NBUF):
            pltpu.make_async_copy(
                rows_v.at[b], agg_sh.at[dst_v.at[0]], ssem.at[b]).wait()

        @pl.when(c == 0)
        def _():
            def dwait(i, carry):
                pltpu.make_async_copy(
                    ones_v, deg_sh.at[dst_v.at[0]], dsem).wait()
                return carry

            lax.fori_loop(0, K, dwait, 0)

        plsc.subcore_barrier()

        for t in range(STRIPE // CH):
            off = s * STRIPE + t * CH
            pltpu.sync_copy(agg_sh.at[pl.ds(off, CH)], rows_v.at[0])
            pltpu.sync_copy(rows_v.at[0], agg_out.at[c, pl.ds(off, CH)])

        @pl.when(c == 0)
        def _():
            for t in range(STRIPE // CH):
                off = s * STRIPE + t * CH
                pltpu.sync_copy(deg_sh.at[pl.ds(off, CH)], degst_v)
                pltpu.sync_copy(degst_v, deg_out.at[pl.ds(off, CH)])

    return run(x2, src2, dst_r, za, zd, ones)


def _tc_body(x_ref, a_ref, d_ref, ws_ref, wn_ref, bs_ref, bn_ref, o_ref):
    xb = x_ref[...]
    deg = jnp.maximum(d_ref[...], 1.0)
    aggn = a_ref[...] / deg
    h = (jnp.dot(xb, ws_ref[...], preferred_element_type=jnp.float32)
         + jnp.dot(aggn, wn_ref[...], preferred_element_type=jnp.float32)
         + bs_ref[...] + bn_ref[...])
    o_ref[...] = jnp.maximum(h, 0.0)


def kernel(x, edge_index, W_self, b_self, W_neigh, b_neigh):
    src = edge_index[0].astype(jnp.int32)
    dst = edge_index[1].astype(jnp.int32)
    pad_e = E_PAD - E
    src = jnp.concatenate([src, jnp.zeros((pad_e,), jnp.int32)])
    pad_dst = N + (jnp.arange(pad_e, dtype=jnp.int32) % (N_PAD - N))
    dst = jnp.concatenate([dst, pad_dst])
    src_r = src.reshape(NS, K, CH)
    dst_r = dst.reshape(NS, K, CH)
    src2 = jnp.stack([src_r, src_r + N_PAD])
    x_pad = jnp.concatenate([x, jnp.zeros((N_PAD - N, D), jnp.float32)])
    x2 = jnp.concatenate([x_pad[:, :DH], x_pad[:, DH:]], axis=0)
    za = jnp.zeros((CH, DH), jnp.float32)
    zd = jnp.zeros((CH, DW), jnp.float32)
    ones = jnp.ones((CH, DW), jnp.float32)

    agg_p, deg_p = _sc_aggregate(x2, src2, dst_r, za, zd, ones)
    agg = jnp.concatenate([agg_p[0], agg_p[1]], axis=1)
    deg1 = deg_p[:, :1]

    R = 1024
    h = pl.pallas_call(
        _tc_body,
        grid=(N_PAD // R,),
        in_specs=[
            pl.BlockSpec((R, D), lambda i: (i, 0)),
            pl.BlockSpec((R, D), lambda i: (i, 0)),
            pl.BlockSpec((R, 1), lambda i: (i, 0)),
            pl.BlockSpec((D, D), lambda i: (0, 0)),
            pl.BlockSpec((D, D), lambda i: (0, 0)),
            pl.BlockSpec((1, D), lambda i: (0, 0)),
            pl.BlockSpec((1, D), lambda i: (0, 0)),
        ],
        out_specs=pl.BlockSpec((R, D), lambda i: (i, 0)),
        out_shape=jax.ShapeDtypeStruct((N_PAD, D), jnp.float32),
    )(x_pad, agg, deg1, W_self.T, W_neigh.T,
      b_self.reshape(1, D), b_neigh.reshape(1, D))
    return h[:N]
